# Initial kernel scaffold; baseline (speedup 1.0000x reference)
import jax
import jax.numpy as jnp
from jax import lax
from jax.experimental import pallas as pl
from jax.experimental.pallas import tpu as pltpu
from jax.experimental.pallas import tpu_sc as plsc

_N = 50000
_E = 800000
_IRR = 32
_DPER = 3
_AVG = 16.0
_H = 64

_CORES = 2
_TILES = 16
_SUB = 128
_NSUB = 8
_CHUNK = _SUB * _NSUB
_MACROS = 49
_EPT = _CHUNK * _MACROS
_EPAD = _EPT * _TILES
_ROWS = _EPAD // _SUB
_NPT = _N // _TILES
_NQ = _NPT // _SUB
_NTAIL = _NPT - _NQ * _SUB
_MLP_B = 4096


def _mlp_body(r_ref, w1_ref, b1_ref, w2_ref, b2_ref, w3_ref, b3_ref, out_ref):
    i = pl.program_id(0)
    r = r_ref[...]
    h = jax.nn.silu(r * w1_ref[...] + b1_ref[...])
    h = jax.nn.silu(
        jnp.dot(h, w2_ref[...], preferred_element_type=jnp.float32) + b2_ref[...]
    )
    mix = jnp.dot(h, w3_ref[...], preferred_element_type=jnp.float32) + b3_ref[...]
    mix = mix * (1.0 / _AVG)
    eidx = i * _MLP_B + lax.broadcasted_iota(jnp.int32, (_MLP_B, 1), 0)
    mix = jnp.where(eidx < _E, mix, 0.0)
    out_ref[0, :, :] = mix[:, :16]
    out_ref[1, :, :] = mix[:, 16:]


def _sc_body(snd_hbm, rcv_hbm, mix_hbm, nf_hbm, out_hbm,
             snd_v, rcv_v, mix_v, nf_v, zero_v, stage_v, acc_sh, sem):
    c = lax.axis_index("c")
    s = lax.axis_index("s")
    nbase = s * _NPT

    def zfill(q, carry):
        zero_v[q, :] = jnp.zeros((16,), jnp.float32)
        return carry
    lax.fori_loop(0, _SUB, zfill, 0, unroll=8)

    for d in range(_DPER):
        u = d * _CORES + c
        tab0 = u * _N
        off16 = jnp.zeros((16,), jnp.int32) + tab0

        for q in range(_NQ):
            pltpu.sync_copy(zero_v, acc_sh.at[pl.ds(nbase + q * _SUB, _SUB)])
        pltpu.sync_copy(zero_v.at[pl.ds(0, _NTAIL)],
                        acc_sh.at[pl.ds(nbase + _NQ * _SUB, _NTAIL)])
        plsc.subcore_barrier()

        def macro(m, carry):
            row0 = s * (_EPT // _SUB) + m * _NSUB
            pltpu.sync_copy(snd_hbm.at[pl.ds(row0, _NSUB)], snd_v)
            pltpu.sync_copy(rcv_hbm.at[pl.ds(row0, _NSUB)], rcv_v)
            pltpu.sync_copy(mix_hbm.at[pl.ds(c * _EPAD + row0 * _SUB, _CHUNK)],
                            mix_v)

            def offs(j, cr):
                for i in range(_SUB // 16):
                    snd_v[j, pl.ds(i * 16, 16)] = (
                        snd_v[j, pl.ds(i * 16, 16)] + off16)
                return cr
            lax.fori_loop(0, _NSUB, offs, 0)

            cps = [
                pltpu.async_copy(nf_hbm.at[snd_v.at[j]],
                                 nf_v.at[pl.ds(j * _SUB, _SUB)], sem)
                for j in range(_NSUB)
            ]
            for cp in cps:
                cp.wait()

            def mul(e, cr):
                nf_v[e, :] = nf_v[e, :] * mix_v[e, :]
                return cr
            lax.fori_loop(0, _CHUNK, mul, 0, unroll=8)

            for j in range(_NSUB):
                pltpu.sync_copy(nf_v.at[pl.ds(j * _SUB, _SUB)],
                                acc_sh.at[rcv_v.at[j]], add=True)
            return carry
        lax.fori_loop(0, _MACROS, macro, 0)
        plsc.subcore_barrier()

        outbase = tab0 + nbase
        for q in range(_NQ):
            pltpu.sync_copy(acc_sh.at[pl.ds(nbase + q * _SUB, _SUB)], stage_v)
            pltpu.sync_copy(stage_v, out_hbm.at[pl.ds(outbase + q * _SUB, _SUB)])
        pltpu.sync_copy(acc_sh.at[pl.ds(nbase + _NQ * _SUB, _NTAIL)],
                        stage_v.at[pl.ds(0, _NTAIL)])
        pltpu.sync_copy(stage_v.at[pl.ds(0, _NTAIL)],
                        out_hbm.at[pl.ds(outbase + _NQ * _SUB, _NTAIL)])


def kernel(vectors, node_feats, radial_embedding, senders, receivers,
           W1, b1, W2, b2, W3, b3):
    pad = _EPAD - _E
    r_pad = jnp.concatenate(
        [radial_embedding, jnp.zeros((pad, 1), jnp.float32)], axis=0)
    mix6 = pl.pallas_call(
        _mlp_body,
        grid=(_EPAD // _MLP_B,),
        in_specs=[
            pl.BlockSpec((_MLP_B, 1), lambda i: (i, 0)),
            pl.BlockSpec((1, _H), lambda i: (0, 0)),
            pl.BlockSpec((1, _H), lambda i: (0, 0)),
            pl.BlockSpec((_H, _H), lambda i: (0, 0)),
            pl.BlockSpec((1, _H), lambda i: (0, 0)),
            pl.BlockSpec((_H, _IRR), lambda i: (0, 0)),
            pl.BlockSpec((1, _IRR), lambda i: (0, 0)),
        ],
        out_specs=pl.BlockSpec((_CORES, _MLP_B, 16), lambda i: (0, i, 0)),
        out_shape=jax.ShapeDtypeStruct((_CORES, _EPAD, 16), jnp.float32),
    )(r_pad, W1, b1.reshape(1, _H), W2, b2.reshape(1, _H),
      W3, b3.reshape(1, _IRR))
    mix_flat = mix6.reshape(_CORES * _EPAD, 16)

    nf_flat = (node_feats.reshape(_N, _CORES, 16, _DPER)
               .transpose(3, 1, 0, 2)
               .reshape(_DPER * _CORES * _N, 16))
    zpad = jnp.zeros((pad,), jnp.int32)
    snd = jnp.concatenate([senders, zpad]).reshape(_ROWS, _SUB)
    rcv = jnp.concatenate([receivers, zpad]).reshape(_ROWS, _SUB)

    mesh = plsc.VectorSubcoreMesh(core_axis_name="c", subcore_axis_name="s")
    out6 = pl.kernel(
        _sc_body,
        out_type=jax.ShapeDtypeStruct((_DPER * _CORES * _N, 16), jnp.float32),
        mesh=mesh,
        scratch_types=[
            pltpu.VMEM((_NSUB, _SUB), jnp.int32),
            pltpu.VMEM((_NSUB, _SUB), jnp.int32),
            pltpu.VMEM((_CHUNK, 16), jnp.float32),
            pltpu.VMEM((_CHUNK, 16), jnp.float32),
            pltpu.VMEM((_SUB, 16), jnp.float32),
            pltpu.VMEM((_SUB, 16), jnp.float32),
            pltpu.VMEM_SHARED((_N, 16), jnp.float32),
            pltpu.SemaphoreType.DMA,
        ],
    )(snd, rcv, mix_flat, nf_flat)

    out = (out6.reshape(_DPER, _CORES, _N, 16)
           .transpose(2, 1, 3, 0)
           .reshape(_N, _IRR, _DPER))
    return out


# speedup vs baseline: 27.8529x; 27.8529x over previous
import jax
import jax.numpy as jnp
from jax import lax
from jax.experimental import pallas as pl
from jax.experimental.pallas import tpu as pltpu
from jax.experimental.pallas import tpu_sc as plsc

_N = 50000
_E = 800000
_IRR = 32
_DPER = 3
_AVG = 16.0
_H = 64

_CORES = 2
_TILES = 16
_SUB = 128
_NSUB = 8
_CHUNK = _SUB * _NSUB
_MACROS = 49
_EPT = _CHUNK * _MACROS
_EPAD = _EPT * _TILES
_ROWS = _EPAD // _SUB
_NPAD = 50176
_NPT = _NPAD // _TILES
_NQ = _NPT // _SUB
_NTAIL = _NPT - _NQ * _SUB
_MLP_B = 4096


def _mlp_body(r_ref, w1_ref, b1_ref, w2_ref, b2_ref, w3_ref, b3_ref, out_ref):
    i = pl.program_id(0)
    r = r_ref[...]
    h = jax.nn.silu(r * w1_ref[...] + b1_ref[...])
    h = jax.nn.silu(
        jnp.dot(h, w2_ref[...], preferred_element_type=jnp.float32) + b2_ref[...]
    )
    mix = jnp.dot(h, w3_ref[...], preferred_element_type=jnp.float32) + b3_ref[...]
    mix = mix * (1.0 / _AVG)
    eidx = i * _MLP_B + lax.broadcasted_iota(jnp.int32, (_MLP_B, 1), 0)
    mix = jnp.where(eidx < _E, mix, 0.0)
    out_ref[0, :, :] = mix[:, :16]
    out_ref[1, :, :] = mix[:, 16:]


def _sc_body(snd_hbm, rcv_hbm, mix_hbm, nf_hbm, out_hbm,
             snd_v, rcv_v, mix_v, nf_v, zero_v, stage_v, acc_sh, sem):
    c = lax.axis_index("c")
    s = lax.axis_index("s")
    nbase = s * _NPT

    def zfill(q, carry):
        zero_v[q, :] = jnp.zeros((16,), jnp.float32)
        return carry
    lax.fori_loop(0, _SUB, zfill, 0, unroll=8)

    for d in range(_DPER):
        u = d * _CORES + c
        tab0 = u * _NPAD
        off16 = jnp.zeros((16,), jnp.int32) + tab0

        for q in range(_NQ):
            pltpu.sync_copy(zero_v, acc_sh.at[pl.ds(nbase + q * _SUB, _SUB)])
        pltpu.sync_copy(zero_v.at[pl.ds(0, _NTAIL)],
                        acc_sh.at[pl.ds(nbase + _NQ * _SUB, _NTAIL)])
        plsc.subcore_barrier()

        def macro(m, carry):
            row0 = s * (_EPT // _SUB) + m * _NSUB
            pltpu.sync_copy(snd_hbm.at[pl.ds(row0, _NSUB)], snd_v)
            pltpu.sync_copy(rcv_hbm.at[pl.ds(row0, _NSUB)], rcv_v)
            pltpu.sync_copy(mix_hbm.at[pl.ds(c * _EPAD + row0 * _SUB, _CHUNK)],
                            mix_v)

            def offs(j, cr):
                for i in range(_SUB // 16):
                    snd_v[j, pl.ds(i * 16, 16)] = (
                        snd_v[j, pl.ds(i * 16, 16)] + off16)
                return cr
            lax.fori_loop(0, _NSUB, offs, 0)

            cps = [
                pltpu.async_copy(nf_hbm.at[snd_v.at[j]],
                                 nf_v.at[pl.ds(j * _SUB, _SUB)], sem)
                for j in range(_NSUB)
            ]
            for cp in cps:
                cp.wait()

            def mul(e, cr):
                nf_v[e, :] = nf_v[e, :] * mix_v[e, :]
                return cr
            lax.fori_loop(0, _CHUNK, mul, 0, unroll=8)

            for j in range(_NSUB):
                pltpu.sync_copy(nf_v.at[pl.ds(j * _SUB, _SUB)],
                                acc_sh.at[rcv_v.at[j]], add=True)
            return carry
        lax.fori_loop(0, _MACROS, macro, 0)
        plsc.subcore_barrier()

        outbase = tab0 + nbase
        for q in range(_NQ):
            pltpu.sync_copy(acc_sh.at[pl.ds(nbase + q * _SUB, _SUB)], stage_v)
            pltpu.sync_copy(stage_v, out_hbm.at[pl.ds(outbase + q * _SUB, _SUB)])
        pltpu.sync_copy(acc_sh.at[pl.ds(nbase + _NQ * _SUB, _NTAIL)],
                        stage_v.at[pl.ds(0, _NTAIL)])
        pltpu.sync_copy(stage_v.at[pl.ds(0, _NTAIL)],
                        out_hbm.at[pl.ds(outbase + _NQ * _SUB, _NTAIL)])


def kernel(vectors, node_feats, radial_embedding, senders, receivers,
           W1, b1, W2, b2, W3, b3):
    pad = _EPAD - _E
    r_pad = jnp.concatenate(
        [radial_embedding, jnp.zeros((pad, 1), jnp.float32)], axis=0)
    mix6 = pl.pallas_call(
        _mlp_body,
        grid=(_EPAD // _MLP_B,),
        in_specs=[
            pl.BlockSpec((_MLP_B, 1), lambda i: (i, 0)),
            pl.BlockSpec((1, _H), lambda i: (0, 0)),
            pl.BlockSpec((1, _H), lambda i: (0, 0)),
            pl.BlockSpec((_H, _H), lambda i: (0, 0)),
            pl.BlockSpec((1, _H), lambda i: (0, 0)),
            pl.BlockSpec((_H, _IRR), lambda i: (0, 0)),
            pl.BlockSpec((1, _IRR), lambda i: (0, 0)),
        ],
        out_specs=pl.BlockSpec((_CORES, _MLP_B, 16), lambda i: (0, i, 0)),
        out_shape=jax.ShapeDtypeStruct((_CORES, _EPAD, 16), jnp.float32),
    )(r_pad, W1, b1.reshape(1, _H), W2, b2.reshape(1, _H),
      W3, b3.reshape(1, _IRR))
    mix_flat = mix6.reshape(_CORES * _EPAD, 16)

    nf_pad = jnp.concatenate(
        [node_feats, jnp.zeros((_NPAD - _N, _IRR, _DPER), jnp.float32)], axis=0)
    nf_flat = (nf_pad.reshape(_NPAD, _CORES, 16, _DPER)
               .transpose(3, 1, 0, 2)
               .reshape(_DPER * _CORES * _NPAD, 16))
    zpad = jnp.zeros((pad,), jnp.int32)
    snd = jnp.concatenate([senders, zpad]).reshape(_ROWS, _SUB)
    rcv = jnp.concatenate([receivers, zpad]).reshape(_ROWS, _SUB)

    mesh = plsc.VectorSubcoreMesh(core_axis_name="c", subcore_axis_name="s")
    out6 = pl.kernel(
        _sc_body,
        out_type=jax.ShapeDtypeStruct((_DPER * _CORES * _NPAD, 16), jnp.float32),
        mesh=mesh,
        compiler_params=pltpu.CompilerParams(use_tc_tiling_on_sc=False),
        scratch_types=[
            pltpu.VMEM((_NSUB, _SUB), jnp.int32),
            pltpu.VMEM((_NSUB, _SUB), jnp.int32),
            pltpu.VMEM((_CHUNK, 16), jnp.float32),
            pltpu.VMEM((_CHUNK, 16), jnp.float32),
            pltpu.VMEM((_SUB, 16), jnp.float32),
            pltpu.VMEM((_SUB, 16), jnp.float32),
            pltpu.VMEM_SHARED((_NPAD, 16), jnp.float32),
            pltpu.SemaphoreType.DMA,
        ],
    )(snd, rcv, mix_flat, nf_flat)

    out = (out6.reshape(_DPER, _CORES, _NPAD, 16)
           .transpose(2, 1, 3, 0)
           .reshape(_NPAD, _IRR, _DPER))
    return out[:_N]
